# Initial kernel scaffold; baseline (speedup 1.0000x reference)
import jax
import jax.numpy as jnp
from jax.experimental import pallas as pl


def kernel(x, edge_index, params):
    raise NotImplementedError("write your pallas kernel here")


# speedup vs baseline: 27.6752x; 27.6752x over previous
import functools

import jax
import jax.numpy as jnp
from jax import lax
from jax.experimental import pallas as pl
from jax.experimental.pallas import tpu as pltpu
from jax.experimental.pallas import tpu_sc as plsc

N = 10000
E = 320000
IN_C = 128
HID = 128
OUT = 64
HEADS = 4

NC = 2
NS = 16
NW = NC * NS
OWN = 320
N2 = NW * OWN
DUMP = OWN
LROWS = OWN + 8

CAP = 16384
EC = 128
SCAN = 512

BR = 512
GRID = N2 // BR

_SC_PARAMS = pltpu.CompilerParams(needs_layout_passes=False)


def _iota16():
    return lax.broadcasted_iota(jnp.int32, (16,), 0)


def _c16(v):
    return jnp.full((16,), v, jnp.int32)


def _ln(t, g, b):
    mu = jnp.mean(t, axis=-1, keepdims=True)
    var = jnp.mean((t - mu) ** 2, axis=-1, keepdims=True)
    return (t - mu) * lax.rsqrt(var + 1e-5) * g + b


def _gelu_exact(t):
    return 0.5 * t * (1.0 + lax.erf(t * 0.7071067811865476))


def _mm(a, b):
    return jnp.dot(a, b, precision=lax.Precision.HIGHEST)


def _rowspec(w):
    return pl.BlockSpec((BR, w), lambda i: (i, 0))


def _fullspec(shape):
    nd = len(shape)
    return pl.BlockSpec(shape, lambda i, _nd=nd: (0,) * _nd)


def _tc_call(body, in_arrays, out_shapes, row_widths_in, row_widths_out):
    in_specs = []
    for a, w in zip(in_arrays, row_widths_in):
        in_specs.append(_fullspec(a.shape) if w is None else _rowspec(w))
    out_specs = [_rowspec(w) for w in row_widths_out]
    return pl.pallas_call(
        body,
        grid=(GRID,),
        in_specs=in_specs,
        out_specs=out_specs if len(out_specs) > 1 else out_specs[0],
        out_shape=(out_shapes if len(out_shapes) > 1 else out_shapes[0]),
    )(*in_arrays)


def _emit_tables(z, als, ald, mod, dout, ztab, srcsc, dstsc):
    if dout == HID:
        ztab[...] = z
        srcsc[...] = jnp.concatenate(
            [als[:, :4], mod, jnp.zeros((BR, 3), jnp.float32)], axis=-1)
    else:
        ztab[...] = jnp.concatenate(
            [z, als[:, :4], mod, jnp.zeros((BR, 128 - dout - 5), jnp.float32)],
            axis=-1)
        srcsc[...] = jnp.zeros((BR, 8), jnp.float32)
    dstsc[...] = jnp.concatenate(
        [ald[:, :4], jnp.zeros((BR, 4), jnp.float32)], axis=-1)


def _k_pre_body(x, pe_w1, pe_b1, pe_w2, pe_b2, in_w, in_b, ln_g, ln_b,
                cw, cb, a_s, a_d, ztab, srcsc, dstsc, pini, hout):
    xb = x[...]
    p1 = jnp.maximum(_mm(xb, pe_w1[...]) + pe_b1[...], 0.0)
    pv = _mm(p1, pe_w2[...]) + pe_b2[...]
    p_init = jax.nn.sigmoid(pv[:, 0:1])
    u = 1.0 - 2.0 * jnp.abs(p_init - 0.5)
    mod = 1.0 + 0.25 * (u * u)
    h = _gelu_exact(_ln(_mm(xb, in_w[...]) + in_b[...], ln_g[...], ln_b[...]))
    z = _mm(h, cw[...]) + cb[...]
    als = _mm(z, a_s[...])
    ald = _mm(z, a_d[...])
    _emit_tables(z, als, ald, mod, HID, ztab, srcsc, dstsc)
    pini[...] = jnp.concatenate(
        [p_init, jnp.zeros((BR, 7), jnp.float32)], axis=-1)
    hout[...] = h


def _k_mid_body(dout, out_raw, s, hprev, pinin, rm, lng, lnb,
                cw, cb, a_s, a_d, ztab, srcsc, dstsc, hout):
    rec = 1.0 / (s[...] + 1e-16)
    norm = out_raw[...] * _mm(rec, rm[...])
    hn = _gelu_exact(_ln(norm, lng[...], lnb[...]))
    h = hn + hprev[...]
    p_init = pinin[:, 0:1]
    u = 1.0 - 2.0 * jnp.abs(p_init - 0.5)
    mod = 1.0 + 0.25 * (u * u)
    z = _mm(h, cw[...]) + cb[...]
    als = _mm(z, a_s[...])
    ald = _mm(z, a_d[...])
    _emit_tables(z, als, ald, mod, dout, ztab, srcsc, dstsc)
    hout[...] = h


def _k_post_body(out_raw, s, hres, pinin, rm, lng, lnb, res_w, res_b,
                 pq_w, pk_wT, protoT, proto, pv_w,
                 g_w1a, g_w1b, g_b1, g_w2, g_b2,
                 cls_w1, cls_b1, cls_w2, cls_b2, logits):
    rec = 1.0 / (s[...] + 1e-16)
    norm = out_raw[...] * _mm(rec, rm[...])
    hn = _gelu_exact(_ln(norm, lng[...], lnb[...]))
    h = hn + (_mm(hres[...], res_w[...]) + res_b[...])
    p_init = pinin[:, 0:1]
    u = 1.0 - 2.0 * jnp.abs(p_init - 0.5)
    q = _mm(h, pq_w[...])
    kT = _mm(pk_wT[...], protoT[...])
    lg = _mm(q, kT) * 0.125
    m = jnp.max(lg, axis=-1, keepdims=True)
    ex = jnp.exp(lg - m)
    attn = ex / jnp.sum(ex, axis=-1, keepdims=True)
    v = _mm(proto[...], pv_w[...])
    h = h + u * _mm(attn, v)
    g1 = jnp.maximum(_mm(h, g_w1a[...]) + p_init * g_w1b[...] + g_b1[...], 0.0)
    gate = jax.nn.sigmoid(_mm(g1, g_w2[...]) + g_b2[...])
    h = h * gate
    cl = _gelu_exact(_mm(h, cls_w1[...]) + cls_b1[...])
    logits[...] = _mm(cl, cls_w2[...]) + cls_b2[...]


_MESH = plsc.VectorSubcoreMesh(core_axis_name="c", subcore_axis_name="s",
                               num_cores=NC, num_subcores=NS)


def _wid():
    return lax.axis_index("c") * NS + lax.axis_index("s")


def _sc_partition(src_hbm, dst_hbm, lsrc_hbm, ldst_hbm, cnt_hbm,
                  sbuf, dbuf, srcl, dstl, cntv, sem1, sem2):
    wid = _wid()
    lo = wid * OWN
    hi = lo + OWN

    def fill(i, _):
        srcl[pl.ds(i * 16, 16)] = jnp.zeros((16,), jnp.int32)
        dstl[pl.ds(i * 16, 16)] = _c16(DUMP)
        return 0

    lax.fori_loop(0, CAP // 16, fill, 0)

    def chunk(ci, cnt):
        base = ci * SCAN
        cp1 = pltpu.make_async_copy(src_hbm.at[pl.ds(base, SCAN)], sbuf, sem1)
        cp2 = pltpu.make_async_copy(dst_hbm.at[pl.ds(base, SCAN)], dbuf, sem2)
        cp1.start()
        cp2.start()
        cp1.wait()
        cp2.wait()

        def grp(g, cnt):
            d = dbuf[pl.ds(g * 16, 16)]
            m = (d >= lo) & (d < hi)
            csum = plsc.cumsum(m.astype(jnp.int32))
            pos = jnp.where(m, csum + (cnt - 1), CAP)
            plsc.store_scatter(dstl, [pos], d - lo)
            sv = sbuf[pl.ds(g * 16, 16)]
            plsc.store_scatter(srcl, [pos], sv)
            return cnt + csum[15]

        return lax.fori_loop(0, SCAN // 16, grp, cnt)

    cnt = lax.fori_loop(0, E // SCAN, chunk, jnp.int32(0))
    padded = ((cnt + EC - 1) // EC) * EC

    def setc(i, _):
        cntv[pl.ds(i * 16, 16)] = _c16(1) * padded
        return 0

    lax.fori_loop(0, 8, setc, 0)
    pltpu.sync_copy(cntv, cnt_hbm.at[pl.ds(wid * 128, 128)])
    pltpu.sync_copy(srcl.at[pl.ds(0, CAP)], lsrc_hbm.at[pl.ds(wid * CAP, CAP)])
    pltpu.sync_copy(dstl.at[pl.ds(0, CAP)], ldst_hbm.at[pl.ds(wid * CAP, CAP)])


def _make_sc_partition():
    return pl.kernel(
        _sc_partition,
        out_type=[
            jax.ShapeDtypeStruct((NW * CAP,), jnp.int32),
            jax.ShapeDtypeStruct((NW * CAP,), jnp.int32),
            jax.ShapeDtypeStruct((NW * 128,), jnp.int32),
        ],
        mesh=_MESH,
        compiler_params=_SC_PARAMS,
        scratch_types=[
            pltpu.VMEM((SCAN,), jnp.int32),
            pltpu.VMEM((SCAN,), jnp.int32),
            pltpu.VMEM((CAP + 16,), jnp.int32),
            pltpu.VMEM((CAP + 16,), jnp.int32),
            pltpu.VMEM((128,), jnp.int32),
            pltpu.SemaphoreType.DMA,
            pltpu.SemaphoreType.DMA,
        ],
    )


def _sc_edge(dout, hd, sep_scalars,
             ztab_hbm, srcsc_hbm, dstsc_hbm, lsrc_hbm, ldst_hbm, cnt_hbm,
             out_hbm, s_hbm, srcl, dstl, aldloc, sloc, outloc, zbuf, wbuf,
             idxb, valb, cntv, semz, sems):
    wid = _wid()
    lo = wid * OWN
    hvec = hd // 16

    pltpu.sync_copy(cnt_hbm.at[pl.ds(wid * 128, 128)], cntv)
    pltpu.sync_copy(lsrc_hbm.at[pl.ds(wid * CAP, CAP)],
                    srcl.at[pl.ds(0, CAP)])
    pltpu.sync_copy(ldst_hbm.at[pl.ds(wid * CAP, CAP)],
                    dstl.at[pl.ds(0, CAP)])
    pltpu.sync_copy(dstsc_hbm.at[pl.ds(lo * 8, OWN * 8)],
                    aldloc.at[pl.ds(0, OWN * 8)])

    def zout(i, _):
        outloc[pl.ds(i * 16, 16)] = jnp.zeros((16,), jnp.float32)
        return 0

    lax.fori_loop(0, (LROWS * dout) // 16, zout, 0)

    def zs(i, _):
        sloc[pl.ds(i * 16, 16)] = jnp.zeros((16,), jnp.float32)
        return 0

    lax.fori_loop(0, (LROWS * 8) // 16, zs, 0)
    for j in range(4):
        aldloc[pl.ds(OWN * 8 + j * 16, 16)] = jnp.zeros((16,), jnp.float32)

    mycnt = cntv[pl.ds(0, 16)][0]
    nchunks = mycnt // EC

    def chunk(ci, _):
        off = ci * EC
        cpz = pltpu.make_async_copy(
            ztab_hbm.at[srcl.at[pl.ds(off, EC)]], zbuf, semz)
        cpz.start()
        iota = _iota16()
        if sep_scalars:
            for g in range(EC // 16):
                srcv = srcl[pl.ds(off + g * 16, 16)]
                b8 = srcv * 8
                for k in range(5):
                    idxb[k][pl.ds(g * 16, 16)] = b8 + k
            cps = [pltpu.make_async_copy(srcsc_hbm.at[idxb[k]], valb[k], sems)
                   for k in range(5)]
            for cp in cps:
                cp.start()
            for cp in cps:
                cp.wait()
        cpz.wait()

        for g in range(EC // 16):
            rloc = iota + g * 16
            dv = dstl[pl.ds(off + g * 16, 16)]
            if sep_scalars:
                modv = valb[4][pl.ds(g * 16, 16)]
            else:
                modv = plsc.load_gather(zbuf, [rloc, _c16(dout + 4)])
            for h in range(HEADS):
                if sep_scalars:
                    alsh = valb[h][pl.ds(g * 16, 16)]
                else:
                    alsh = plsc.load_gather(zbuf, [rloc, _c16(dout + h)])
                aldh = plsc.load_gather(aldloc, [dv * 8 + h])
                t = alsh + aldh
                lk = jnp.maximum(t, 0.2 * t)
                w = jnp.exp(lk * modv)
                plsc.addupdate_scatter(sloc, [dv * 8 + h], w)
                plsc.store_scatter(wbuf, [rloc * 8 + h], w)

        def acc(e, _):
            dl = dstl[pl.ds(off + e, 16)][0]
            wv = wbuf[pl.ds(e * 8, 16)]
            base = dl * dout
            for h in range(HEADS):
                wsc = wv[h]
                for jj in range(hvec):
                    col = h * hd + jj * 16
                    zv = zbuf[e, pl.ds(col, 16)]
                    plsc.addupdate(outloc.at[pl.ds(base + col, 16)], zv * wsc)
            return 0

        lax.fori_loop(0, EC, acc, 0)
        return 0

    lax.fori_loop(0, nchunks, chunk, 0)
    pltpu.sync_copy(outloc.at[pl.ds(0, OWN * dout)],
                    out_hbm.at[pl.ds(lo * dout, OWN * dout)])
    pltpu.sync_copy(sloc.at[pl.ds(0, OWN * 8)],
                    s_hbm.at[pl.ds(lo * 8, OWN * 8)])


def _make_sc_edge(dout, hd, sep_scalars):
    def body(ztab, srcsc, dstsc, lsrc, ldst, cnt, out, s,
             srcl, dstl, aldloc, sloc, outloc, zbuf, wbuf,
             i0, i1, i2, i3, i4, v0, v1, v2, v3, v4, cntv, semz, sems):
        _sc_edge(dout, hd, sep_scalars, ztab, srcsc, dstsc, lsrc, ldst, cnt,
                 out, s, srcl, dstl, aldloc, sloc, outloc, zbuf, wbuf,
                 [i0, i1, i2, i3, i4], [v0, v1, v2, v3, v4], cntv, semz, sems)

    return pl.kernel(
        body,
        out_type=[
            jax.ShapeDtypeStruct((N2 * dout,), jnp.float32),
            jax.ShapeDtypeStruct((N2 * 8,), jnp.float32),
        ],
        mesh=_MESH,
        compiler_params=_SC_PARAMS,
        scratch_types=[
            pltpu.VMEM((CAP,), jnp.int32),
            pltpu.VMEM((CAP + 16,), jnp.int32),
            pltpu.VMEM((LROWS * 8,), jnp.float32),
            pltpu.VMEM((LROWS * 8,), jnp.float32),
            pltpu.VMEM((LROWS * dout,), jnp.float32),
            pltpu.VMEM((EC, 128), jnp.float32),
            pltpu.VMEM((EC * 8 + 16,), jnp.float32),
            pltpu.VMEM((EC,), jnp.int32),
            pltpu.VMEM((EC,), jnp.int32),
            pltpu.VMEM((EC,), jnp.int32),
            pltpu.VMEM((EC,), jnp.int32),
            pltpu.VMEM((EC,), jnp.int32),
            pltpu.VMEM((EC,), jnp.float32),
            pltpu.VMEM((EC,), jnp.float32),
            pltpu.VMEM((EC,), jnp.float32),
            pltpu.VMEM((EC,), jnp.float32),
            pltpu.VMEM((EC,), jnp.float32),
            pltpu.VMEM((128,), jnp.int32),
            pltpu.SemaphoreType.DMA,
            pltpu.SemaphoreType.DMA,
        ],
    )


def _head_mats(a_s, a_d, hd):
    eye = jnp.eye(HEADS, dtype=jnp.float32)
    asm = (a_s[:, :, None] * eye[:, None, :]).reshape(HEADS * hd, HEADS)
    adm = (a_d[:, :, None] * eye[:, None, :]).reshape(HEADS * hd, HEADS)
    pad = jnp.zeros((HEADS * hd, 4), jnp.float32)
    asm = jnp.concatenate([asm, pad], axis=1)
    adm = jnp.concatenate([adm, pad], axis=1)
    rm = (eye[:, :, None] * jnp.ones((hd,), jnp.float32)).reshape(
        HEADS, HEADS * hd)
    rm = jnp.concatenate(
        [rm, jnp.zeros((4, HEADS * hd), jnp.float32)], axis=0)
    return asm, adm, rm


def kernel(x, edge_index, params):
    p = params
    x_pad = jnp.concatenate(
        [x, jnp.zeros((N2 - N, IN_C), jnp.float32)], axis=0)
    src = edge_index[0].reshape(E)
    dst = edge_index[1].reshape(E)

    row1 = lambda a: a.reshape(1, -1)
    pe_w2 = jnp.concatenate(
        [p['pe_w2'], jnp.zeros((64, 7), jnp.float32)], axis=1)
    pe_b2 = jnp.broadcast_to(p['pe_b2'].reshape(1, 1), (1, 8))

    as0, ad0, rm0 = _head_mats(p['asrc0'], p['adst0'], HID // HEADS)
    as1, ad1, _ = _head_mats(p['asrc1'], p['adst1'], HID // HEADS)
    as2, ad2, rm2 = _head_mats(p['asrc2'], p['adst2'], OUT // HEADS)

    ztab0, srcsc0, dstsc0, pini, h0 = _tc_call(
        _k_pre_body,
        [x_pad, p['pe_w1'], row1(p['pe_b1']), pe_w2, pe_b2,
         p['in_w'], row1(p['in_b']), row1(p['in_ln_g']), row1(p['in_ln_b']),
         p['cw0'], row1(p['cb0']), as0, ad0],
        [jax.ShapeDtypeStruct((N2, 128), jnp.float32),
         jax.ShapeDtypeStruct((N2, 8), jnp.float32),
         jax.ShapeDtypeStruct((N2, 8), jnp.float32),
         jax.ShapeDtypeStruct((N2, 8), jnp.float32),
         jax.ShapeDtypeStruct((N2, HID), jnp.float32)],
        [IN_C] + [None] * 12,
        [128, 8, 8, 8, HID])

    lsrc, ldst, cnt = _make_sc_partition()(src, dst)

    edge128 = _make_sc_edge(HID, HID // HEADS, True)
    edge64 = _make_sc_edge(OUT, OUT // HEADS, False)

    out0f, s0f = edge128(ztab0, srcsc0.reshape(N2 * 8), dstsc0.reshape(N2 * 8),
                         lsrc, ldst, cnt)
    out0 = out0f.reshape(N2, HID)
    s0 = s0f.reshape(N2, 8)

    ztab1, srcsc1, dstsc1, h1 = _tc_call(
        functools.partial(_k_mid_body, HID),
        [out0, s0, h0, pini, rm0, row1(p['lng0']), row1(p['lnb0']),
         p['cw1'], row1(p['cb1']), as1, ad1],
        [jax.ShapeDtypeStruct((N2, 128), jnp.float32),
         jax.ShapeDtypeStruct((N2, 8), jnp.float32),
         jax.ShapeDtypeStruct((N2, 8), jnp.float32),
         jax.ShapeDtypeStruct((N2, HID), jnp.float32)],
        [HID, 8, HID, 8] + [None] * 7,
        [128, 8, 8, HID])

    out1f, s1f = edge128(ztab1, srcsc1.reshape(N2 * 8), dstsc1.reshape(N2 * 8),
                         lsrc, ldst, cnt)
    out1 = out1f.reshape(N2, HID)
    s1 = s1f.reshape(N2, 8)

    ztab2, srcsc2, dstsc2, h2 = _tc_call(
        functools.partial(_k_mid_body, OUT),
        [out1, s1, h1, pini, rm0, row1(p['lng1']), row1(p['lnb1']),
         p['cw2'], row1(p['cb2']), as2, ad2],
        [jax.ShapeDtypeStruct((N2, 128), jnp.float32),
         jax.ShapeDtypeStruct((N2, 8), jnp.float32),
         jax.ShapeDtypeStruct((N2, 8), jnp.float32),
         jax.ShapeDtypeStruct((N2, HID), jnp.float32)],
        [HID, 8, HID, 8] + [None] * 7,
        [128, 8, 8, HID])

    out2f, s2f = edge64(ztab2, srcsc2.reshape(N2 * 8), dstsc2.reshape(N2 * 8),
                        lsrc, ldst, cnt)
    out2 = out2f.reshape(N2, OUT)
    s2 = s2f.reshape(N2, 8)

    cls_w2 = jnp.concatenate(
        [p['cls_w2'], jnp.zeros((32, 7), jnp.float32)], axis=1)
    cls_b2 = jnp.broadcast_to(p['cls_b2'].reshape(1, 1), (1, 8))

    logits_pad = _tc_call(
        _k_post_body,
        [out2, s2, h2, pini, rm2, row1(p['lng2']), row1(p['lnb2']),
         p['res_w'], row1(p['res_b']),
         p['pq_w'], p['pk_w'].T, p['proto'].T, p['proto'], p['pv_w'],
         p['g_w1'][:OUT], row1(p['g_w1'][OUT]), row1(p['g_b1']),
         p['g_w2'], row1(p['g_b2']),
         p['cls_w1'], row1(p['cls_b1']), cls_w2, cls_b2],
        [jax.ShapeDtypeStruct((N2, 8), jnp.float32)],
        [OUT, 8, HID, 8] + [None] * 19,
        [8])

    return logits_pad[:N, 0:1]
